# Initial kernel scaffold; baseline (speedup 1.0000x reference)
import jax
import jax.numpy as jnp
from jax.experimental import pallas as pl


def kernel(x, edge_index, batch, params):
    raise NotImplementedError("write your pallas kernel here")


# speedup vs baseline: 1.0035x; 1.0035x over previous
import functools

import jax
import jax.numpy as jnp
import numpy as np
from jax import lax
from jax.experimental import pallas as pl
from jax.experimental.pallas import tpu as pltpu

_N = 10000
_E = 320000
_D = 128
_H = 8
_C = 16
_G = 64
_BR = 1000
_INV_SQRT2 = np.float32(1.0 / np.sqrt(2.0))


def _gelu(x):
    return 0.5 * x * (1.0 + lax.erf(x * _INV_SQRT2))


def _proj_body(h_ref, w_ref, b_ref, q_ref, k_ref, v_ref, s_ref):
    y = jnp.dot(h_ref[...], w_ref[...], preferred_element_type=jnp.float32)
    y = y + b_ref[...]
    q_ref[...] = y[:, 0:128]
    k_ref[...] = y[:, 128:256]
    v_ref[...] = y[:, 256:384]
    s_ref[...] = y[:, 384:512]


def _proj(h, wcat, bcat):
    nb = _N // _BR
    return pl.pallas_call(
        _proj_body,
        grid=(nb,),
        in_specs=[
            pl.BlockSpec((_BR, _D), lambda i: (i, 0)),
            pl.BlockSpec((_D, 512), lambda i: (0, 0)),
            pl.BlockSpec((1, 512), lambda i: (0, 0)),
        ],
        out_specs=[pl.BlockSpec((_BR, _D), lambda i: (i, 0))] * 4,
        out_shape=[jax.ShapeDtypeStruct((_N, _D), jnp.float32)] * 4,
    )(h, wcat, bcat)


def _combine_body(m_ref, xr_ref, wb_ref, g_ref, b_ref, o_ref):
    m = m_ref[...]
    xr = xr_ref[...]
    w1 = wb_ref[0:1, :]
    w2 = wb_ref[1:2, :]
    w3 = wb_ref[2:3, :]
    lin = jnp.sum(m * w1 + xr * w2 + (m - xr) * w3, axis=-1, keepdims=True)
    beta = jax.nn.sigmoid(lin)
    out = beta * xr + (1.0 - beta) * m
    mu = jnp.mean(out, axis=-1, keepdims=True)
    var = jnp.mean((out - mu) ** 2, axis=-1, keepdims=True)
    out = (out - mu) * lax.rsqrt(var + 1e-5) * g_ref[...] + b_ref[...]
    o_ref[...] = _gelu(out)


def _combine(m, xr, wb3, ln_g, ln_b):
    nb = _N // _BR
    return pl.pallas_call(
        _combine_body,
        grid=(nb,),
        in_specs=[
            pl.BlockSpec((_BR, _D), lambda i: (i, 0)),
            pl.BlockSpec((_BR, _D), lambda i: (i, 0)),
            pl.BlockSpec((3, _D), lambda i: (0, 0)),
            pl.BlockSpec((1, _D), lambda i: (0, 0)),
            pl.BlockSpec((1, _D), lambda i: (0, 0)),
        ],
        out_specs=pl.BlockSpec((_BR, _D), lambda i: (i, 0)),
        out_shape=jax.ShapeDtypeStruct((_N, _D), jnp.float32),
    )(m, xr, wb3, ln_g, ln_b)


def _pool_body(h_ref, bt_ref, w1_ref, b1_ref, w2_ref, b2_ref,
               y_ref, acc_ref, cnt_ref, *, nb):
    i = pl.program_id(0)

    @pl.when(i == 0)
    def _init():
        acc_ref[...] = jnp.zeros_like(acc_ref)
        cnt_ref[...] = jnp.zeros_like(cnt_ref)

    bt = bt_ref[0, 0, :]
    gids = lax.broadcasted_iota(jnp.int32, (_G, _BR), 0)
    mask = (bt[None, :] == gids).astype(jnp.float32)
    acc_ref[...] += jnp.dot(mask, h_ref[...], preferred_element_type=jnp.float32)
    cnt_ref[...] += jnp.broadcast_to(
        jnp.sum(mask, axis=1, keepdims=True), (_G, _D))

    @pl.when(i == nb - 1)
    def _head():
        g = acc_ref[...] / jnp.maximum(cnt_ref[...], 1.0)
        hm = jnp.dot(g, w1_ref[...], preferred_element_type=jnp.float32)
        hm = _gelu(hm + b1_ref[...])
        y = jnp.dot(hm, w2_ref[...], preferred_element_type=jnp.float32)
        y_ref[...] = jnp.broadcast_to(y + b2_ref[...], (_G, _D))


def _pool_head(h, batch3, w1, b1, w2, b2):
    nb = _N // _BR
    return pl.pallas_call(
        functools.partial(_pool_body, nb=nb),
        grid=(nb,),
        in_specs=[
            pl.BlockSpec((_BR, _D), lambda i: (i, 0)),
            pl.BlockSpec((1, 1, _BR), lambda i: (i, 0, 0)),
            pl.BlockSpec((_D, _D), lambda i: (0, 0)),
            pl.BlockSpec((1, _D), lambda i: (0, 0)),
            pl.BlockSpec((_D, 1), lambda i: (0, 0)),
            pl.BlockSpec((1, 1), lambda i: (0, 0)),
        ],
        out_specs=pl.BlockSpec((_G, _D), lambda i: (0, 0)),
        out_shape=jax.ShapeDtypeStruct((_G, _D), jnp.float32),
        scratch_shapes=[
            pltpu.VMEM((_G, _D), jnp.float32),
            pltpu.VMEM((_G, _D), jnp.float32),
        ],
    )(h, batch3, w1, b1, w2, b2)


def _edge_attention(q, k, v, src, dst):
    qh = q.reshape(_N, _H, _C)
    kh = k.reshape(_N, _H, _C)
    vh = v.reshape(_N, _H, _C)
    alpha = jnp.sum(qh[dst] * kh[src], axis=-1) / np.float32(np.sqrt(_C))
    amax = jax.ops.segment_max(alpha, dst, num_segments=_N)
    amax = jnp.where(jnp.isfinite(amax), amax, 0.0)
    ae = jnp.exp(alpha - amax[dst])
    s = jax.ops.segment_sum(ae, dst, num_segments=_N)
    alpha = ae / (s[dst] + 1e-16)
    msg = vh[src] * alpha[:, :, None]
    return jax.ops.segment_sum(msg, dst, num_segments=_N).reshape(_N, _D)


def kernel(x, edge_index, batch, params):
    src = edge_index[0]
    dst = edge_index[1]
    batch3 = batch.reshape(_N // _BR, 1, _BR)
    h = x
    for l in range(2):
        p = lambda n, _l=l: params[f"l{_l}_{n}"]
        wcat = jnp.concatenate(
            [p("Wq"), p("Wk"), p("Wv"), p("Wskip")], axis=1)
        bcat = jnp.concatenate(
            [p("bq"), p("bk"), p("bv"), p("bskip")], axis=0).reshape(1, 512)
        q, k, v, xr = _proj(h, wcat, bcat)
        m = _edge_attention(q, k, v, src, dst)
        wb3 = p("Wbeta").reshape(3, _D)
        h = _combine(m, xr, wb3,
                     p("ln_g").reshape(1, _D), p("ln_b").reshape(1, _D))
    y = _pool_head(h, batch3, params["mlp_W1"],
                   params["mlp_b1"].reshape(1, _D),
                   params["mlp_W2"], params["mlp_b2"].reshape(1, 1))
    return y[:, 0]
